# baseline (speedup 1.0000x reference)
import functools

import jax
import jax.numpy as jnp
from jax import lax
from jax.experimental import pallas as pl
from jax.experimental.pallas import tpu as pltpu
from jax.experimental.pallas import tpu_sc as plsc

B, N, VOCAB, EMB = 16, 1024, 4096, 192

_NCHUNKS = 4
_CB = B // _NCHUNKS


def _argmax_body(x_ref, tok_ref):
    xb = x_ref[0]
    m = jnp.max(xb, axis=-1, keepdims=True)
    iota = lax.broadcasted_iota(jnp.int32, xb.shape, 1)
    idx = jnp.min(jnp.where(xb == m, iota, VOCAB), axis=-1)
    tok_ref[0, 0] = idx.astype(jnp.int32)


def _argmax_tokens_chunk(x, c):
    return pl.pallas_call(
        _argmax_body,
        grid=(_CB,),
        in_specs=[pl.BlockSpec((1, N, VOCAB), lambda b: (c * _CB + b, 0, 0))],
        out_specs=pl.BlockSpec((1, 1, N), lambda b: (b, 0, 0)),
        out_shape=jax.ShapeDtypeStruct((_CB, 1, N), jnp.int32),
    )(x)


_NC, _NS, _L = 2, 16, 16
_NW = _NC * _NS
_BPW = _CB * N // _NW
_WPB = N // _BPW
_CHUNK = 128
_NGATH = _BPW // _CHUNK


def _make_sc_gather():
    mesh = plsc.VectorSubcoreMesh(core_axis_name="c", subcore_axis_name="s")

    @functools.partial(
        pl.kernel,
        mesh=mesh,
        out_type=jax.ShapeDtypeStruct((_CB, N, EMB), jnp.float32),
        scratch_types=[
            pltpu.VMEM((_NGATH, _CHUNK), jnp.int32),
            pltpu.VMEM((_BPW, EMB), jnp.float32),
            pltpu.SemaphoreType.DMA,
        ],
        compiler_params=pltpu.CompilerParams(use_tc_tiling_on_sc=False),
    )
    def sc_gather(table_hbm, idx_hbm, out_hbm, idx_v, rows_v, sem):
        wid = lax.axis_index("s") * _NC + lax.axis_index("c")
        b = wid // _WPB
        noff = (wid % _WPB) * _BPW
        for j in range(_NGATH):
            pltpu.sync_copy(
                idx_hbm.at[b, 0, pl.ds(noff + j * _CHUNK, _CHUNK)],
                idx_v.at[j],
            )
        copies = []
        for j in range(_NGATH):
            copies.append(
                pltpu.async_copy(
                    table_hbm.at[idx_v.at[j]],
                    rows_v.at[pl.ds(j * _CHUNK, _CHUNK)],
                    sem,
                )
            )
        for c in copies:
            c.wait()
        pltpu.sync_copy(rows_v, out_hbm.at[b, pl.ds(noff, _BPW)])

    return sc_gather


_SC_GATHER_CACHE = []


def kernel(x, dictionary):
    if not _SC_GATHER_CACHE:
        _SC_GATHER_CACHE.append(_make_sc_gather())
    gather = _SC_GATHER_CACHE[0]
    outs = []
    for c in range(_NCHUNKS):
        tokens_c = _argmax_tokens_chunk(x, c)
        outs.append(gather(dictionary, tokens_c))
    return jnp.concatenate(outs, axis=0)


# speedup vs baseline: 1.2626x; 1.2626x over previous
import functools

import jax
import jax.numpy as jnp
from jax import lax
from jax.experimental import pallas as pl
from jax.experimental.pallas import tpu as pltpu
from jax.experimental.pallas import tpu_sc as plsc

B, N, VOCAB, EMB = 16, 1024, 4096, 192
EMBP = 256


def _argmax_body(x_ref, tok_ref):
    xb = x_ref[0]
    m = jnp.max(xb, axis=-1, keepdims=True)
    iota = lax.broadcasted_iota(jnp.int32, xb.shape, 1)
    idx = jnp.min(jnp.where(xb == m, iota, VOCAB), axis=-1)
    tok_ref[0, 0] = idx.astype(jnp.int32)


def _argmax_tokens(x):
    return pl.pallas_call(
        _argmax_body,
        grid=(B,),
        in_specs=[pl.BlockSpec((1, N, VOCAB), lambda b: (b, 0, 0))],
        out_specs=pl.BlockSpec((1, 1, N), lambda b: (b, 0, 0)),
        out_shape=jax.ShapeDtypeStruct((B, 1, N), jnp.int32),
    )(x)


_NC, _NS, _L = 2, 16, 16
_NW = _NC * _NS
_BPW = B * N // _NW
_WPB = N // _BPW
_CHUNK = 128
_NGATH = _BPW // _CHUNK
_NSLOT = 3


def _make_sc_gather():
    mesh = plsc.VectorSubcoreMesh(core_axis_name="c", subcore_axis_name="s")

    @functools.partial(
        pl.kernel,
        mesh=mesh,
        out_type=jax.ShapeDtypeStruct((B, N, EMBP), jnp.float32),
        scratch_types=[
            pltpu.VMEM((_NGATH, _CHUNK), jnp.int32),
            pltpu.VMEM((_NSLOT, _CHUNK, EMBP), jnp.float32),
            pltpu.SemaphoreType.DMA,
        ],
        compiler_params=pltpu.CompilerParams(use_tc_tiling_on_sc=True),
    )
    def sc_gather(table_hbm, idx_hbm, out_hbm, idx_v, rows_v, sem):
        wid = lax.axis_index("s") * _NC + lax.axis_index("c")
        b = wid // _WPB
        noff = (wid % _WPB) * _BPW
        for j in range(_NGATH):
            pltpu.sync_copy(
                idx_hbm.at[b, 0, pl.ds(noff + j * _CHUNK, _CHUNK)],
                idx_v.at[j],
            )
        queue = []
        for j in range(_NGATH):
            s = j % _NSLOT
            if len(queue) == _NSLOT:
                c0, j0, s0 = queue.pop(0)
                c0.wait()
                pltpu.sync_copy(
                    rows_v.at[s0],
                    out_hbm.at[b, pl.ds(noff + j0 * _CHUNK, _CHUNK)],
                )
            queue.append(
                (
                    pltpu.async_copy(
                        table_hbm.at[idx_v.at[j]], rows_v.at[s], sem
                    ),
                    j,
                    s,
                )
            )
        for c0, j0, s0 in queue:
            c0.wait()
            pltpu.sync_copy(
                rows_v.at[s0],
                out_hbm.at[b, pl.ds(noff + j0 * _CHUNK, _CHUNK)],
            )

    return sc_gather


_SC_GATHER_CACHE = []


def kernel(x, dictionary):
    if not _SC_GATHER_CACHE:
        _SC_GATHER_CACHE.append(_make_sc_gather())
    tokens = _argmax_tokens(x)
    dict_p = jnp.pad(dictionary, ((0, 0), (0, EMBP - EMB)))
    out_p = _SC_GATHER_CACHE[0](dict_p, tokens)
    return out_p[:, :, :EMB]


# speedup vs baseline: 1.2762x; 1.0108x over previous
import functools

import jax
import jax.numpy as jnp
from jax import lax
from jax.experimental import pallas as pl
from jax.experimental.pallas import tpu as pltpu
from jax.experimental.pallas import tpu_sc as plsc

B, N, VOCAB, EMB = 16, 1024, 4096, 192
EMBP = 256


_TROW = B * N // 128


def _argmax_body(x_ref, tok_ref):
    xb = x_ref[0]
    m = jnp.max(xb, axis=-1, keepdims=True)
    iota = lax.broadcasted_iota(jnp.int32, xb.shape, 1)
    idx = jnp.min(jnp.where(xb == m, iota, VOCAB), axis=-1)
    tok_ref[...] = idx.astype(jnp.int32).reshape(N // 128, 128)


def _argmax_tokens(x):
    return pl.pallas_call(
        _argmax_body,
        grid=(B,),
        in_specs=[pl.BlockSpec((1, N, VOCAB), lambda b: (b, 0, 0))],
        out_specs=pl.BlockSpec((N // 128, 128), lambda b: (b, 0)),
        out_shape=jax.ShapeDtypeStruct((_TROW, 128), jnp.int32),
    )(x)


_NC, _NS, _L = 2, 16, 16
_NW = _NC * _NS
_BPW = B * N // _NW
_WPB = N // _BPW
_CHUNK = 128
_NGATH = _BPW // _CHUNK
_NSLOT = 3


def _make_sc_gather():
    mesh = plsc.VectorSubcoreMesh(core_axis_name="c", subcore_axis_name="s")

    @functools.partial(
        pl.kernel,
        mesh=mesh,
        out_type=jax.ShapeDtypeStruct((B, N, EMBP), jnp.float32),
        scratch_types=[
            pltpu.VMEM((_NGATH, _CHUNK), jnp.int32),
            pltpu.VMEM((_NSLOT, _CHUNK, EMBP), jnp.float32),
            pltpu.SemaphoreType.DMA,
        ],
        compiler_params=pltpu.CompilerParams(use_tc_tiling_on_sc=True),
    )
    def sc_gather(table_hbm, idx_hbm, out_hbm, idx_v, rows_v, sem):
        wid = lax.axis_index("s") * _NC + lax.axis_index("c")
        b = wid // _WPB
        noff = (wid % _WPB) * _BPW
        pltpu.sync_copy(idx_hbm.at[pl.ds(wid * _NGATH, _NGATH)], idx_v)
        queue = []
        for j in range(_NGATH):
            s = j % _NSLOT
            if len(queue) == _NSLOT:
                c0, j0, s0 = queue.pop(0)
                c0.wait()
                pltpu.sync_copy(
                    rows_v.at[s0],
                    out_hbm.at[b, pl.ds(noff + j0 * _CHUNK, _CHUNK)],
                )
            queue.append(
                (
                    pltpu.async_copy(
                        table_hbm.at[idx_v.at[j]], rows_v.at[s], sem
                    ),
                    j,
                    s,
                )
            )
        for c0, j0, s0 in queue:
            c0.wait()
            pltpu.sync_copy(
                rows_v.at[s0],
                out_hbm.at[b, pl.ds(noff + j0 * _CHUNK, _CHUNK)],
            )

    return sc_gather


_SC_GATHER_CACHE = []


def kernel(x, dictionary):
    if not _SC_GATHER_CACHE:
        _SC_GATHER_CACHE.append(_make_sc_gather())
    tokens = _argmax_tokens(x)
    dict_p = jnp.pad(dictionary, ((0, 0), (0, EMBP - EMB)))
    out_p = _SC_GATHER_CACHE[0](dict_p, tokens)
    return out_p[:, :, :EMB]
